# baseline (speedup 1.0000x reference)
import functools

import jax
import jax.numpy as jnp
from jax import lax
from jax.experimental import pallas as pl
from jax.experimental.pallas import tpu as pltpu
from jax.experimental.pallas import tpu_sc as plsc

N = 10000
H = 128
NG = 64
BLK = 1000
NB = N // BLK

NC, NS = 2, 16
NW = NC * NS
CH = 112
NPAD = 10112
RPT = NPAD // NS


def _relu(v):
    return jnp.maximum(v, 0.0)


def _leaky(v):
    return jnp.maximum(v, 0.01 * v)


def _enc_table_body(xb, w1, b1, w2, b2, e8, lw, lb, h_out, m_out):
    h = jnp.dot(_relu(jnp.dot(xb[...], w1[...], preferred_element_type=jnp.float32) + b1[...]),
                w2[...], preferred_element_type=jnp.float32) + b2[...]
    e = jnp.dot(e8[...], lw[...], preferred_element_type=jnp.float32) + lb[...]
    h_out[...] = h
    m_out[0] = _relu(h + e[0:1, :])
    m_out[1] = _relu(h + e[1:2, :])


def _conv_table_body(hb, aggb, w1, b1, sc, be, w2, b2, e8, lw, lb, h_out, m_out):
    hs = hb[...] + aggb[0] + aggb[1]
    y = _relu(jnp.dot(hs, w1[...], preferred_element_type=jnp.float32) + b1[...]) * sc[...] + be[...]
    z = _leaky(jnp.dot(y, w2[...], preferred_element_type=jnp.float32) + b2[...])
    e = jnp.dot(e8[...], lw[...], preferred_element_type=jnp.float32) + lb[...]
    h_out[...] = z
    m_out[0] = _relu(z + e[0:1, :])
    m_out[1] = _relu(z + e[1:2, :])


def _final_body(hb, aggb, w1, b1, sc, be, w2, b2, batchb, mw1, mb1, mw2r, mb2,
                out, pool_acc, cnt_acc):
    i = pl.program_id(0)

    @pl.when(i == 0)
    def _():
        pool_acc[...] = jnp.zeros_like(pool_acc)
        cnt_acc[...] = jnp.zeros_like(cnt_acc)

    hs = hb[...] + aggb[0] + aggb[1]
    y = _relu(jnp.dot(hs, w1[...], preferred_element_type=jnp.float32) + b1[...]) * sc[...] + be[...]
    z = _leaky(jnp.dot(y, w2[...], preferred_element_type=jnp.float32) + b2[...])

    b = batchb[0, 0, :]
    gids = lax.broadcasted_iota(jnp.int32, (NG, BLK), 0)
    oh = (gids == b[None, :]).astype(jnp.float32)
    pool_acc[...] += jnp.dot(oh, z, preferred_element_type=jnp.float32)
    cnt_acc[...] += jnp.broadcast_to(jnp.sum(oh, axis=1, keepdims=True), cnt_acc.shape)

    @pl.when(i == NB - 1)
    def _():
        pooled = pool_acc[...] / jnp.maximum(cnt_acc[...], 1.0)
        z1 = _relu(jnp.dot(pooled, mw1[...], preferred_element_type=jnp.float32) + mb1[...])
        z2 = jnp.sum(z1 * mw2r[...], axis=1, keepdims=True) + mb2[...]
        out[...] = jax.nn.sigmoid(z2)


def _full(shape):
    return pl.BlockSpec(shape, lambda i: tuple(0 for _ in shape))


def _enc_table(xp, w1p, b1, w2, b2, e8, lw, lb):
    return pl.pallas_call(
        _enc_table_body,
        grid=(NB,),
        in_specs=[
            pl.BlockSpec((BLK, 16), lambda i: (i, 0)),
            _full((16, H)), _full((1, H)), _full((H, H)), _full((1, H)),
            _full((8, H)), _full((H, H)), _full((1, H)),
        ],
        out_specs=[
            pl.BlockSpec((BLK, H), lambda i: (i, 0)),
            pl.BlockSpec((2, BLK, H), lambda i: (0, i, 0)),
        ],
        out_shape=[
            jax.ShapeDtypeStruct((N, H), jnp.float32),
            jax.ShapeDtypeStruct((2, N, H), jnp.float32),
        ],
    )(xp, w1p, b1, w2, b2, e8, lw, lb)


def _conv_table(h, agg, w1, b1, sc, be, w2, b2, e8, lw, lb):
    return pl.pallas_call(
        _conv_table_body,
        grid=(NB,),
        in_specs=[
            pl.BlockSpec((BLK, H), lambda i: (i, 0)),
            pl.BlockSpec((2, BLK, H), lambda i: (0, i, 0)),
            _full((H, H)), _full((1, H)), _full((1, H)), _full((1, H)),
            _full((H, H)), _full((1, H)),
            _full((8, H)), _full((H, H)), _full((1, H)),
        ],
        out_specs=[
            pl.BlockSpec((BLK, H), lambda i: (i, 0)),
            pl.BlockSpec((2, BLK, H), lambda i: (0, i, 0)),
        ],
        out_shape=[
            jax.ShapeDtypeStruct((N, H), jnp.float32),
            jax.ShapeDtypeStruct((2, N, H), jnp.float32),
        ],
    )(h, agg, w1, b1, sc, be, w2, b2, e8, lw, lb)


def _final(h, agg, w1, b1, sc, be, w2, b2, batch3, mw1, mb1, mw2r, mb2):
    return pl.pallas_call(
        _final_body,
        grid=(NB,),
        in_specs=[
            pl.BlockSpec((BLK, H), lambda i: (i, 0)),
            pl.BlockSpec((2, BLK, H), lambda i: (0, i, 0)),
            _full((H, H)), _full((1, H)), _full((1, H)), _full((1, H)),
            _full((H, H)), _full((1, H)),
            pl.BlockSpec((1, 1, BLK), lambda i: (i, 0, 0)),
            _full((H, NG)), _full((1, NG)), _full((1, NG)), _full((1, 1)),
        ],
        out_specs=pl.BlockSpec((NG, 1), lambda i: (0, 0)),
        out_shape=jax.ShapeDtypeStruct((NG, 1), jnp.float32),
        scratch_shapes=[
            pltpu.VMEM((NG, H), jnp.float32),
            pltpu.VMEM((NG, H), jnp.float32),
        ],
    )(h, agg, w1, b1, sc, be, w2, b2, batch3, mw1, mb1, mw2r, mb2)


NBUF = 3
NPH = 3


def _gidx_body(srcb, typb, out):
    out[...] = srcb[...] + typb[...] * N


def _gidx(srcp, typp):
    rows = srcp.shape[0] // 128
    rb = rows // 5
    src2, typ2 = srcp.reshape(rows, 128), typp.reshape(rows, 128)
    out = pl.pallas_call(
        _gidx_body,
        grid=(5,),
        in_specs=[pl.BlockSpec((rb, 128), lambda i: (i, 0)),
                  pl.BlockSpec((rb, 128), lambda i: (i, 0))],
        out_specs=pl.BlockSpec((rb, 128), lambda i: (i, 0)),
        out_shape=jax.ShapeDtypeStruct((rows, 128), jnp.int32),
    )(src2, typ2)
    return out.reshape(-1)


def _sc_agg_body(ept, m_hbm, gidx_hbm, dst_hbm, out_hbm,
                 gidxv, dstv, rowbufs, acc, gsems, ssems):
    ep2 = ept // NPH
    k = ep2 // CH
    c = lax.axis_index("c")
    s = lax.axis_index("s")
    w = c * NS + s
    ebase = w * ept

    zv = jnp.zeros((16,), jnp.float32)

    def zrow(r, carry):
        for l in range(8):
            rowbufs[0][r, pl.ds(l * 16, 16)] = zv
        return carry

    lax.fori_loop(0, CH, zrow, 0)

    row0 = s * RPT
    nfull, rem = RPT // CH, RPT % CH
    for p in range(nfull):
        pltpu.sync_copy(rowbufs[0], acc.at[pl.ds(row0 + p * CH, CH)])
    if rem:
        pltpu.sync_copy(rowbufs[0].at[pl.ds(0, rem)], acc.at[pl.ds(row0 + nfull * CH, rem)])

    def fire_gather(j, b):
        pltpu.async_copy(m_hbm.at[gidxv.at[pl.ds(j * CH, CH)]], rowbufs[b], gsems[b])

    def wait_gather(j, b):
        pltpu.make_async_copy(m_hbm.at[gidxv.at[pl.ds(j * CH, CH)]],
                              rowbufs[b], gsems[b]).wait()

    def fire_scatter(j, b):
        pltpu.async_copy(rowbufs[b], acc.at[dstv.at[pl.ds(j * CH, CH)]],
                         ssems[b], add=True)

    def wait_scatter(j, b):
        pltpu.make_async_copy(rowbufs[b], acc.at[dstv.at[pl.ds(j * CH, CH)]],
                              ssems[b]).wait()

    pltpu.sync_copy(gidx_hbm.at[pl.ds(ebase, ep2)], gidxv)
    pltpu.sync_copy(dst_hbm.at[pl.ds(ebase, ep2)], dstv)
    plsc.subcore_barrier()

    for ph in range(NPH):
        for b in range(NBUF - 1):
            fire_gather(b, b)

        def step(g, carry):
            for b in range(NBUF):
                j = g * NBUF + b
                nb = (b + NBUF - 1) % NBUF
                jn = j + NBUF - 1
                if b == 0:
                    @pl.when(g > 0)
                    def _():
                        wait_scatter(j - 1, nb)
                else:
                    wait_scatter(j - 1, nb)

                @pl.when(jn < k)
                def _():
                    fire_gather(jn, nb)

                wait_gather(j, b)
                fire_scatter(j, b)
            return carry

        lax.fori_loop(0, k // NBUF, step, 0)
        wait_scatter(k - 1, (k - 1) % NBUF)

        if ph < NPH - 1:
            pltpu.sync_copy(gidx_hbm.at[pl.ds(ebase + (ph + 1) * ep2, ep2)], gidxv)
            pltpu.sync_copy(dst_hbm.at[pl.ds(ebase + (ph + 1) * ep2, ep2)], dstv)

    plsc.subcore_barrier()
    pltpu.sync_copy(acc.at[pl.ds(row0, RPT)], out_hbm.at[c, pl.ds(row0, RPT)])


def _sc_agg(m2d, gidxp, dstp):
    ept = gidxp.shape[0] // NW
    return pl.kernel(
        functools.partial(_sc_agg_body, ept),
        out_type=jax.ShapeDtypeStruct((NC, NPAD, H), jnp.float32),
        mesh=plsc.VectorSubcoreMesh(core_axis_name="c", subcore_axis_name="s",
                                    num_cores=NC, num_subcores=NS),
        scratch_types=[
            pltpu.VMEM((ept // NPH,), jnp.int32),
            pltpu.VMEM((ept // NPH,), jnp.int32),
            [pltpu.VMEM((CH, H), jnp.float32) for _ in range(NBUF)],
            pltpu.VMEM_SHARED((NPAD, H), jnp.float32),
            [pltpu.SemaphoreType.DMA for _ in range(NBUF)],
            [pltpu.SemaphoreType.DMA for _ in range(NBUF)],
        ],
    )(m2d, gidxp, dstp)


def kernel(x, edge_index, edge_type, batch,
           enc_w1, enc_b1, enc_w2, enc_b2, edge_w, edge_b,
           c1_lin_w, c1_lin_b, c1_w1, c1_b1, c1_g, c1_be, c1_w2, c1_b2,
           c2_lin_w, c2_lin_b, c2_w1, c2_b1, c2_g, c2_be, c2_w2, c2_b2,
           m_w1, m_b1, m_w2, m_b2):
    e = edge_index.shape[1]
    in_f = x.shape[1]

    xp = jnp.pad(x, ((0, 0), (0, 16 - in_f)))
    w1p = jnp.pad(enc_w1, ((0, 16 - in_f), (0, 0)))
    r = lambda v: v.reshape(1, -1)
    e2 = jnp.stack([edge_b, edge_w[0] + edge_b])
    e8 = jnp.pad(e2, ((0, 6), (0, 0)))
    bn_inv = 1.0 / jnp.sqrt(1.0 + 1e-5)
    sc1, sc2 = r(c1_g * bn_inv), r(c2_g * bn_inv)

    kk = -(-e // (NW * CH))
    kk += (-kk) % (NBUF * NPH)
    epad = NW * kk * CH
    pad = epad - e
    srcp = jnp.concatenate([edge_index[0], jnp.zeros((pad,), jnp.int32)])
    typp = jnp.concatenate([edge_type, jnp.zeros((pad,), jnp.int32)])
    dstp = jnp.concatenate([edge_index[1], jnp.full((pad,), N, jnp.int32)])
    batch3 = batch.reshape(NB, 1, BLK)

    gidxp = _gidx(srcp, typp)
    h0, m1 = _enc_table(xp, w1p, r(enc_b1), enc_w2, r(enc_b2),
                        e8, c1_lin_w, r(c1_lin_b))
    agg1 = _sc_agg(m1.reshape(2 * N, H), gidxp, dstp)
    h1, m2 = _conv_table(h0, agg1, c1_w1, r(c1_b1), sc1, r(c1_be),
                         c1_w2, r(c1_b2), e8, c2_lin_w, r(c2_lin_b))
    agg2 = _sc_agg(m2.reshape(2 * N, H), gidxp, dstp)
    return _final(h1, agg2, c2_w1, r(c2_b1), sc2, r(c2_be), c2_w2, r(c2_b2),
                  batch3, m_w1, r(m_b1), r(m_w2[:, 0]), m_b2.reshape(1, 1))


# speedup vs baseline: 1.9989x; 1.9989x over previous
import functools

import jax
import jax.numpy as jnp
from jax import lax
from jax.experimental import pallas as pl
from jax.experimental.pallas import tpu as pltpu
from jax.experimental.pallas import tpu_sc as plsc

N = 10000
H = 128
NG = 64
BLK = 1000
NB = N // BLK

NC, NS = 2, 16
NW = NC * NS
CH = 112
NPAD = 10112
RPT = NPAD // NS


def _relu(v):
    return jnp.maximum(v, 0.0)


def _leaky(v):
    return jnp.maximum(v, 0.01 * v)


def _enc_table_body(xb, w1, b1, w2, b2, e8, lw, lb, h_out, m_out):
    h = jnp.dot(_relu(jnp.dot(xb[...], w1[...], preferred_element_type=jnp.float32) + b1[...]),
                w2[...], preferred_element_type=jnp.float32) + b2[...]
    e = jnp.dot(e8[...], lw[...], preferred_element_type=jnp.float32) + lb[...]
    h_out[...] = h
    m_out[0] = _relu(h + e[0:1, :])
    m_out[1] = _relu(h + e[1:2, :])


def _conv_table_body(hb, aggb, w1, b1, sc, be, w2, b2, e8, lw, lb, h_out, m_out):
    hs = hb[...] + aggb[0] + aggb[1]
    y = _relu(jnp.dot(hs, w1[...], preferred_element_type=jnp.float32) + b1[...]) * sc[...] + be[...]
    z = _leaky(jnp.dot(y, w2[...], preferred_element_type=jnp.float32) + b2[...])
    e = jnp.dot(e8[...], lw[...], preferred_element_type=jnp.float32) + lb[...]
    h_out[...] = z
    m_out[0] = _relu(z + e[0:1, :])
    m_out[1] = _relu(z + e[1:2, :])


def _final_body(hb, aggb, w1, b1, sc, be, w2, b2, batchb, mw1, mb1, mw2r, mb2,
                out, pool_acc, cnt_acc):
    i = pl.program_id(0)

    @pl.when(i == 0)
    def _():
        pool_acc[...] = jnp.zeros_like(pool_acc)
        cnt_acc[...] = jnp.zeros_like(cnt_acc)

    hs = hb[...] + aggb[0] + aggb[1]
    y = _relu(jnp.dot(hs, w1[...], preferred_element_type=jnp.float32) + b1[...]) * sc[...] + be[...]
    z = _leaky(jnp.dot(y, w2[...], preferred_element_type=jnp.float32) + b2[...])

    b = batchb[0, 0, :]
    gids = lax.broadcasted_iota(jnp.int32, (NG, BLK), 0)
    oh = (gids == b[None, :]).astype(jnp.float32)
    pool_acc[...] += jnp.dot(oh, z, preferred_element_type=jnp.float32)
    cnt_acc[...] += jnp.broadcast_to(jnp.sum(oh, axis=1, keepdims=True), cnt_acc.shape)

    @pl.when(i == NB - 1)
    def _():
        pooled = pool_acc[...] / jnp.maximum(cnt_acc[...], 1.0)
        z1 = _relu(jnp.dot(pooled, mw1[...], preferred_element_type=jnp.float32) + mb1[...])
        z2 = jnp.sum(z1 * mw2r[...], axis=1, keepdims=True) + mb2[...]
        out[...] = jax.nn.sigmoid(z2)


def _full(shape):
    return pl.BlockSpec(shape, lambda i: tuple(0 for _ in shape))


def _enc_table(xp, w1p, b1, w2, b2, e8, lw, lb):
    return pl.pallas_call(
        _enc_table_body,
        grid=(NB,),
        in_specs=[
            pl.BlockSpec((BLK, 16), lambda i: (i, 0)),
            _full((16, H)), _full((1, H)), _full((H, H)), _full((1, H)),
            _full((8, H)), _full((H, H)), _full((1, H)),
        ],
        out_specs=[
            pl.BlockSpec((BLK, H), lambda i: (i, 0)),
            pl.BlockSpec((2, BLK, H), lambda i: (0, i, 0)),
        ],
        out_shape=[
            jax.ShapeDtypeStruct((N, H), jnp.float32),
            jax.ShapeDtypeStruct((2, N, H), jnp.float32),
        ],
    )(xp, w1p, b1, w2, b2, e8, lw, lb)


def _conv_table(h, agg, w1, b1, sc, be, w2, b2, e8, lw, lb):
    return pl.pallas_call(
        _conv_table_body,
        grid=(NB,),
        in_specs=[
            pl.BlockSpec((BLK, H), lambda i: (i, 0)),
            pl.BlockSpec((2, BLK, H), lambda i: (0, i, 0)),
            _full((H, H)), _full((1, H)), _full((1, H)), _full((1, H)),
            _full((H, H)), _full((1, H)),
            _full((8, H)), _full((H, H)), _full((1, H)),
        ],
        out_specs=[
            pl.BlockSpec((BLK, H), lambda i: (i, 0)),
            pl.BlockSpec((2, BLK, H), lambda i: (0, i, 0)),
        ],
        out_shape=[
            jax.ShapeDtypeStruct((N, H), jnp.float32),
            jax.ShapeDtypeStruct((2, N, H), jnp.float32),
        ],
    )(h, agg, w1, b1, sc, be, w2, b2, e8, lw, lb)


def _final(h, agg, w1, b1, sc, be, w2, b2, batch3, mw1, mb1, mw2r, mb2):
    return pl.pallas_call(
        _final_body,
        grid=(NB,),
        in_specs=[
            pl.BlockSpec((BLK, H), lambda i: (i, 0)),
            pl.BlockSpec((2, BLK, H), lambda i: (0, i, 0)),
            _full((H, H)), _full((1, H)), _full((1, H)), _full((1, H)),
            _full((H, H)), _full((1, H)),
            pl.BlockSpec((1, 1, BLK), lambda i: (i, 0, 0)),
            _full((H, NG)), _full((1, NG)), _full((1, NG)), _full((1, 1)),
        ],
        out_specs=pl.BlockSpec((NG, 1), lambda i: (0, 0)),
        out_shape=jax.ShapeDtypeStruct((NG, 1), jnp.float32),
        scratch_shapes=[
            pltpu.VMEM((NG, H), jnp.float32),
            pltpu.VMEM((NG, H), jnp.float32),
        ],
    )(h, agg, w1, b1, sc, be, w2, b2, batch3, mw1, mb1, mw2r, mb2)


NBUF = 2
NPH = 1


def _gidx_body(srcb, typb, out):
    out[...] = srcb[...] + typb[...] * N


def _gidx(srcp, typp):
    rows = srcp.shape[0] // 128
    rb = rows // 5
    src2, typ2 = srcp.reshape(rows, 128), typp.reshape(rows, 128)
    out = pl.pallas_call(
        _gidx_body,
        grid=(5,),
        in_specs=[pl.BlockSpec((rb, 128), lambda i: (i, 0)),
                  pl.BlockSpec((rb, 128), lambda i: (i, 0))],
        out_specs=pl.BlockSpec((rb, 128), lambda i: (i, 0)),
        out_shape=jax.ShapeDtypeStruct((rows, 128), jnp.int32),
    )(src2, typ2)
    return out.reshape(-1)


def _sc_agg_body(ept, m_hbm, gidx_hbm, dst_hbm, out_hbm,
                 gidxv, dstv, rowbufs, acc, gsems, ssems):
    ep2 = ept // NPH
    k = ep2 // CH
    c = lax.axis_index("c")
    s = lax.axis_index("s")
    w = c * NS + s
    ebase = w * ept

    zv = jnp.zeros((16,), jnp.float32)

    def zrow(r, carry):
        for l in range(8):
            rowbufs[0][r, pl.ds(l * 16, 16)] = zv
        return carry

    lax.fori_loop(0, CH, zrow, 0)

    row0 = s * RPT
    nfull, rem = RPT // CH, RPT % CH
    for p in range(nfull):
        pltpu.sync_copy(rowbufs[0], acc.at[pl.ds(row0 + p * CH, CH)])
    if rem:
        pltpu.sync_copy(rowbufs[0].at[pl.ds(0, rem)], acc.at[pl.ds(row0 + nfull * CH, rem)])

    def fire_gather(j, b):
        pltpu.async_copy(m_hbm.at[gidxv.at[pl.ds(j * CH, CH)]], rowbufs[b], gsems[b])

    def wait_gather(j, b):
        pltpu.make_async_copy(m_hbm.at[gidxv.at[pl.ds(j * CH, CH)]],
                              rowbufs[b], gsems[b]).wait()

    def fire_scatter(j, b):
        pltpu.async_copy(rowbufs[b], acc.at[dstv.at[pl.ds(j * CH, CH)]],
                         ssems[b], add=True)

    def wait_scatter(j, b):
        pltpu.make_async_copy(rowbufs[b], acc.at[dstv.at[pl.ds(j * CH, CH)]],
                              ssems[b]).wait()

    pltpu.sync_copy(gidx_hbm.at[pl.ds(ebase, ep2)], gidxv)
    pltpu.sync_copy(dst_hbm.at[pl.ds(ebase, ep2)], dstv)
    plsc.subcore_barrier()

    for ph in range(NPH):
        for b in range(NBUF - 1):
            fire_gather(b, b)

        def step(g, carry):
            for b in range(NBUF):
                j = g * NBUF + b
                nb = (b + NBUF - 1) % NBUF
                jn = j + NBUF - 1
                if b == 0:
                    @pl.when(g > 0)
                    def _():
                        wait_scatter(j - 1, nb)
                else:
                    wait_scatter(j - 1, nb)

                @pl.when(jn < k)
                def _():
                    fire_gather(jn, nb)

                wait_gather(j, b)
                fire_scatter(j, b)
            return carry

        lax.fori_loop(0, k // NBUF, step, 0)
        wait_scatter(k - 1, (k - 1) % NBUF)

        if ph < NPH - 1:
            pltpu.sync_copy(gidx_hbm.at[pl.ds(ebase + (ph + 1) * ep2, ep2)], gidxv)
            pltpu.sync_copy(dst_hbm.at[pl.ds(ebase + (ph + 1) * ep2, ep2)], dstv)

    plsc.subcore_barrier()
    pltpu.sync_copy(acc.at[pl.ds(row0, RPT)], out_hbm.at[c, pl.ds(row0, RPT)])


def _sc_agg(m2d, gidxp, dstp):
    ept = gidxp.shape[0] // NW
    return pl.kernel(
        functools.partial(_sc_agg_body, ept),
        out_type=jax.ShapeDtypeStruct((NC, NPAD, H), jnp.float32),
        mesh=plsc.VectorSubcoreMesh(core_axis_name="c", subcore_axis_name="s",
                                    num_cores=NC, num_subcores=NS),
        scratch_types=[
            pltpu.VMEM((ept // NPH,), jnp.int32),
            pltpu.VMEM((ept // NPH,), jnp.int32),
            [pltpu.VMEM((CH, H), jnp.float32) for _ in range(NBUF)],
            pltpu.VMEM_SHARED((NPAD, H), jnp.float32),
            [pltpu.SemaphoreType.DMA for _ in range(NBUF)],
            [pltpu.SemaphoreType.DMA for _ in range(NBUF)],
        ],
    )(m2d, gidxp, dstp)


def kernel(x, edge_index, edge_type, batch,
           enc_w1, enc_b1, enc_w2, enc_b2, edge_w, edge_b,
           c1_lin_w, c1_lin_b, c1_w1, c1_b1, c1_g, c1_be, c1_w2, c1_b2,
           c2_lin_w, c2_lin_b, c2_w1, c2_b1, c2_g, c2_be, c2_w2, c2_b2,
           m_w1, m_b1, m_w2, m_b2):
    e = edge_index.shape[1]
    in_f = x.shape[1]

    xp = jnp.pad(x, ((0, 0), (0, 16 - in_f)))
    w1p = jnp.pad(enc_w1, ((0, 16 - in_f), (0, 0)))
    r = lambda v: v.reshape(1, -1)
    e2 = jnp.stack([edge_b, edge_w[0] + edge_b])
    e8 = jnp.pad(e2, ((0, 6), (0, 0)))
    bn_inv = 1.0 / jnp.sqrt(1.0 + 1e-5)
    sc1, sc2 = r(c1_g * bn_inv), r(c2_g * bn_inv)

    kk = -(-e // (NW * CH))
    kk += (-kk) % (NBUF * NPH)
    epad = NW * kk * CH
    pad = epad - e
    ar = jnp.arange(pad, dtype=jnp.int32)
    srcp = jnp.concatenate([edge_index[0], ar % N])
    typp = jnp.concatenate([edge_type, jnp.zeros((pad,), jnp.int32)])
    dstp = jnp.concatenate([edge_index[1], N + ar % (NPAD - N)])
    batch3 = batch.reshape(NB, 1, BLK)

    gidxp = _gidx(srcp, typp)
    h0, m1 = _enc_table(xp, w1p, r(enc_b1), enc_w2, r(enc_b2),
                        e8, c1_lin_w, r(c1_lin_b))
    agg1 = _sc_agg(m1.reshape(2 * N, H), gidxp, dstp)
    h1, m2 = _conv_table(h0, agg1, c1_w1, r(c1_b1), sc1, r(c1_be),
                         c1_w2, r(c1_b2), e8, c2_lin_w, r(c2_lin_b))
    agg2 = _sc_agg(m2.reshape(2 * N, H), gidxp, dstp)
    return _final(h1, agg2, c2_w1, r(c2_b1), sc2, r(c2_be), c2_w2, r(c2_b2),
                  batch3, m_w1, r(m_b1), r(m_w2[:, 0]), m_b2.reshape(1, 1))


# speedup vs baseline: 2.0779x; 1.0395x over previous
import functools

import jax
import jax.numpy as jnp
from jax import lax
from jax.experimental import pallas as pl
from jax.experimental.pallas import tpu as pltpu
from jax.experimental.pallas import tpu_sc as plsc

N = 10000
H = 128
NG = 64
BLK = 1000
NB = N // BLK

NC, NS = 2, 16
NW = NC * NS
CH = 112
NPAD = 10112
RPT = NPAD // NS


def _relu(v):
    return jnp.maximum(v, 0.0)


def _leaky(v):
    return jnp.maximum(v, 0.01 * v)


def _enc_table_body(xb, w1, b1, w2, b2, e8, lw, lb, h_out, m_out):
    h = jnp.dot(_relu(jnp.dot(xb[...], w1[...], preferred_element_type=jnp.float32) + b1[...]),
                w2[...], preferred_element_type=jnp.float32) + b2[...]
    e = jnp.dot(e8[...], lw[...], preferred_element_type=jnp.float32) + lb[...]
    h_out[...] = h
    m_out[0] = _relu(h + e[0:1, :])
    m_out[1] = _relu(h + e[1:2, :])


def _conv_table_body(hb, aggb, w1, b1, sc, be, w2, b2, e8, lw, lb, h_out, m_out):
    hs = hb[...] + aggb[0] + aggb[1]
    y = _relu(jnp.dot(hs, w1[...], preferred_element_type=jnp.float32) + b1[...]) * sc[...] + be[...]
    z = _leaky(jnp.dot(y, w2[...], preferred_element_type=jnp.float32) + b2[...])
    e = jnp.dot(e8[...], lw[...], preferred_element_type=jnp.float32) + lb[...]
    h_out[...] = z
    m_out[0] = _relu(z + e[0:1, :])
    m_out[1] = _relu(z + e[1:2, :])


def _final_body(hb, aggb, w1, b1, sc, be, w2, b2, batchb, mw1, mb1, mw2r, mb2,
                out, pool_acc, cnt_acc):
    i = pl.program_id(0)

    @pl.when(i == 0)
    def _():
        pool_acc[...] = jnp.zeros_like(pool_acc)
        cnt_acc[...] = jnp.zeros_like(cnt_acc)

    hs = hb[...] + aggb[0] + aggb[1]
    y = _relu(jnp.dot(hs, w1[...], preferred_element_type=jnp.float32) + b1[...]) * sc[...] + be[...]
    z = _leaky(jnp.dot(y, w2[...], preferred_element_type=jnp.float32) + b2[...])

    b = batchb[0, 0, :]
    gids = lax.broadcasted_iota(jnp.int32, (NG, BLK), 0)
    oh = (gids == b[None, :]).astype(jnp.float32)
    pool_acc[...] += jnp.dot(oh, z, preferred_element_type=jnp.float32)
    cnt_acc[...] += jnp.broadcast_to(jnp.sum(oh, axis=1, keepdims=True), cnt_acc.shape)

    @pl.when(i == NB - 1)
    def _():
        pooled = pool_acc[...] / jnp.maximum(cnt_acc[...], 1.0)
        z1 = _relu(jnp.dot(pooled, mw1[...], preferred_element_type=jnp.float32) + mb1[...])
        z2 = jnp.sum(z1 * mw2r[...], axis=1, keepdims=True) + mb2[...]
        out[...] = jax.nn.sigmoid(z2)


def _full(shape):
    return pl.BlockSpec(shape, lambda i: tuple(0 for _ in shape))


def _enc_table(xp, w1p, b1, w2, b2, e8, lw, lb):
    return pl.pallas_call(
        _enc_table_body,
        grid=(NB,),
        in_specs=[
            pl.BlockSpec((BLK, 16), lambda i: (i, 0)),
            _full((16, H)), _full((1, H)), _full((H, H)), _full((1, H)),
            _full((8, H)), _full((H, H)), _full((1, H)),
        ],
        out_specs=[
            pl.BlockSpec((BLK, H), lambda i: (i, 0)),
            pl.BlockSpec((2, BLK, H), lambda i: (0, i, 0)),
        ],
        out_shape=[
            jax.ShapeDtypeStruct((N, H), jnp.float32),
            jax.ShapeDtypeStruct((2, N, H), jnp.float32),
        ],
    )(xp, w1p, b1, w2, b2, e8, lw, lb)


def _conv_table(h, agg, w1, b1, sc, be, w2, b2, e8, lw, lb):
    return pl.pallas_call(
        _conv_table_body,
        grid=(NB,),
        in_specs=[
            pl.BlockSpec((BLK, H), lambda i: (i, 0)),
            pl.BlockSpec((2, BLK, H), lambda i: (0, i, 0)),
            _full((H, H)), _full((1, H)), _full((1, H)), _full((1, H)),
            _full((H, H)), _full((1, H)),
            _full((8, H)), _full((H, H)), _full((1, H)),
        ],
        out_specs=[
            pl.BlockSpec((BLK, H), lambda i: (i, 0)),
            pl.BlockSpec((2, BLK, H), lambda i: (0, i, 0)),
        ],
        out_shape=[
            jax.ShapeDtypeStruct((N, H), jnp.float32),
            jax.ShapeDtypeStruct((2, N, H), jnp.float32),
        ],
    )(h, agg, w1, b1, sc, be, w2, b2, e8, lw, lb)


def _final(h, agg, w1, b1, sc, be, w2, b2, batch3, mw1, mb1, mw2r, mb2):
    return pl.pallas_call(
        _final_body,
        grid=(NB,),
        in_specs=[
            pl.BlockSpec((BLK, H), lambda i: (i, 0)),
            pl.BlockSpec((2, BLK, H), lambda i: (0, i, 0)),
            _full((H, H)), _full((1, H)), _full((1, H)), _full((1, H)),
            _full((H, H)), _full((1, H)),
            pl.BlockSpec((1, 1, BLK), lambda i: (i, 0, 0)),
            _full((H, NG)), _full((1, NG)), _full((1, NG)), _full((1, 1)),
        ],
        out_specs=pl.BlockSpec((NG, 1), lambda i: (0, 0)),
        out_shape=jax.ShapeDtypeStruct((NG, 1), jnp.float32),
        scratch_shapes=[
            pltpu.VMEM((NG, H), jnp.float32),
            pltpu.VMEM((NG, H), jnp.float32),
        ],
    )(h, agg, w1, b1, sc, be, w2, b2, batch3, mw1, mb1, mw2r, mb2)


NBUF = 2


def _gidx_body(eib, typb, out):
    out[...] = eib[0] + typb[...] * N


def _gidx(eir, etr):
    rows = etr.shape[0]
    return pl.pallas_call(
        _gidx_body,
        grid=(1,),
        in_specs=[pl.BlockSpec((1, rows, 128), lambda i: (0, 0, 0)),
                  pl.BlockSpec((rows, 128), lambda i: (0, 0))],
        out_specs=pl.BlockSpec((rows, 128), lambda i: (0, 0)),
        out_shape=jax.ShapeDtypeStruct((rows, 128), jnp.int32),
    )(eir, etr)


def _sc_agg_body(ept, nlast, m_hbm, gidx_hbm, ei_hbm, out_hbm,
                 gidxv, dstv, rowbufs, acc, gsems, ssems):
    k = ept // CH
    c = lax.axis_index("c")
    s = lax.axis_index("s")
    w = c * NS + s
    ebase = w * ept

    zv = jnp.zeros((16,), jnp.float32)

    def zrow(r, carry):
        for l in range(8):
            rowbufs[0][r, pl.ds(l * 16, 16)] = zv
        return carry

    lax.fori_loop(0, CH, zrow, 0)

    row0 = s * RPT
    nfull, rem = RPT // CH, RPT % CH
    for p in range(nfull):
        pltpu.sync_copy(rowbufs[0], acc.at[pl.ds(row0 + p * CH, CH)])
    if rem:
        pltpu.sync_copy(rowbufs[0].at[pl.ds(0, rem)], acc.at[pl.ds(row0 + nfull * CH, rem)])

    def fire_gather(j, b):
        pltpu.async_copy(m_hbm.at[gidxv.at[pl.ds(j * CH, CH)]], rowbufs[b], gsems[b])

    def wait_gather(j, b):
        pltpu.make_async_copy(m_hbm.at[gidxv.at[pl.ds(j * CH, CH)]],
                              rowbufs[b], gsems[b]).wait()

    def fire_scatter(j, b):
        pltpu.async_copy(rowbufs[b], acc.at[dstv.at[pl.ds(j * CH, CH)]],
                         ssems[b], add=True)

    def wait_scatter(j, b):
        pltpu.make_async_copy(rowbufs[b], acc.at[dstv.at[pl.ds(j * CH, CH)]],
                              ssems[b]).wait()

    er = (NW - 1) * ept + nlast

    @pl.when(w < NW - 1)
    def _():
        pltpu.sync_copy(gidx_hbm.at[pl.ds(ebase, ept)], gidxv)
        pltpu.sync_copy(ei_hbm.at[pl.ds(er + ebase, ept)], dstv)

    @pl.when(w == NW - 1)
    def _():
        pltpu.sync_copy(gidx_hbm.at[pl.ds(ebase, nlast)], gidxv.at[pl.ds(0, nlast)])
        pltpu.sync_copy(ei_hbm.at[pl.ds(er + ebase, nlast)], dstv.at[pl.ds(0, nlast)])
        lane = jnp.arange(16, dtype=jnp.int32)

        def fill(r, carry):
            off = nlast + r * 16
            gidxv[pl.ds(off, 16)] = lane + r * 16
            dstv[pl.ds(off, 16)] = lane + (r % 7) * 16 + N
            return carry

        lax.fori_loop(0, (ept - nlast) // 16, fill, 0)

    plsc.subcore_barrier()

    for b in range(NBUF - 1):
        fire_gather(b, b)

    def step(g, carry):
        for b in range(NBUF):
            j = g * NBUF + b
            nb = (b + NBUF - 1) % NBUF
            jn = j + NBUF - 1
            if b == 0:
                @pl.when(g > 0)
                def _():
                    wait_scatter(j - 1, nb)
            else:
                wait_scatter(j - 1, nb)

            @pl.when(jn < k)
            def _():
                fire_gather(jn, nb)

            wait_gather(j, b)
            fire_scatter(j, b)
        return carry

    lax.fori_loop(0, k // NBUF, step, 0)
    wait_scatter(k - 1, (k - 1) % NBUF)

    plsc.subcore_barrier()
    pltpu.sync_copy(acc.at[pl.ds(row0, RPT)], out_hbm.at[c, pl.ds(row0, RPT)])


def _sc_agg(m2d, gidxp, ei, ept, nlast):
    return pl.kernel(
        functools.partial(_sc_agg_body, ept, nlast),
        out_type=jax.ShapeDtypeStruct((NC, NPAD, H), jnp.float32),
        mesh=plsc.VectorSubcoreMesh(core_axis_name="c", subcore_axis_name="s",
                                    num_cores=NC, num_subcores=NS),
        scratch_types=[
            pltpu.VMEM((ept,), jnp.int32),
            pltpu.VMEM((ept,), jnp.int32),
            [pltpu.VMEM((CH, H), jnp.float32) for _ in range(NBUF)],
            pltpu.VMEM_SHARED((NPAD, H), jnp.float32),
            [pltpu.SemaphoreType.DMA for _ in range(NBUF)],
            [pltpu.SemaphoreType.DMA for _ in range(NBUF)],
        ],
    )(m2d, gidxp, ei)


def kernel(x, edge_index, edge_type, batch,
           enc_w1, enc_b1, enc_w2, enc_b2, edge_w, edge_b,
           c1_lin_w, c1_lin_b, c1_w1, c1_b1, c1_g, c1_be, c1_w2, c1_b2,
           c2_lin_w, c2_lin_b, c2_w1, c2_b1, c2_g, c2_be, c2_w2, c2_b2,
           m_w1, m_b1, m_w2, m_b2):
    e = edge_index.shape[1]
    in_f = x.shape[1]

    xp = jnp.pad(x, ((0, 0), (0, 16 - in_f)))
    w1p = jnp.pad(enc_w1, ((0, 16 - in_f), (0, 0)))
    r = lambda v: v.reshape(1, -1)
    e2 = jnp.stack([edge_b, edge_w[0] + edge_b])
    e8 = jnp.pad(e2, ((0, 6), (0, 0)))
    bn_inv = 1.0 / jnp.sqrt(1.0 + 1e-5)
    sc1, sc2 = r(c1_g * bn_inv), r(c2_g * bn_inv)

    kk = -(-e // (NW * CH))
    kk += (-kk) % NBUF
    ept = kk * CH
    nlast = e - (NW - 1) * ept
    batch3 = batch.reshape(NB, 1, BLK)

    eir = edge_index.reshape(2, e // 128, 128)
    etr = edge_type.reshape(e // 128, 128)
    gidxp = _gidx(eir, etr).reshape(-1)
    h0, m1 = _enc_table(xp, w1p, r(enc_b1), enc_w2, r(enc_b2),
                        e8, c1_lin_w, r(c1_lin_b))
    agg1 = _sc_agg(m1.reshape(2 * N, H), gidxp, edge_index.reshape(-1), ept, nlast)
    h1, m2 = _conv_table(h0, agg1, c1_w1, r(c1_b1), sc1, r(c1_be),
                         c1_w2, r(c1_b2), e8, c2_lin_w, r(c2_lin_b))
    agg2 = _sc_agg(m2.reshape(2 * N, H), gidxp, edge_index.reshape(-1), ept, nlast)
    return _final(h1, agg2, c2_w1, r(c2_b1), sc2, r(c2_be), c2_w2, r(c2_b2),
                  batch3, m_w1, r(m_b1), r(m_w2[:, 0]), m_b2.reshape(1, 1))
